# Initial kernel scaffold; baseline (speedup 1.0000x reference)
import jax
import jax.numpy as jnp
from jax.experimental import pallas as pl


def kernel(x, sep_token, token_table, pos_table, seg_table):
    raise NotImplementedError("write your pallas kernel here")


# speedup vs baseline: 1.3009x; 1.3009x over previous
import dataclasses
import functools
import math

import jax
import jax.numpy as jnp
from jax import lax
from jax.experimental import pallas as pl
from jax.experimental.pallas import tpu as pltpu
from jax.experimental.pallas import tpu_sc as plsc

B = 4
S = 2048
D = 768
N = B * S
NC, NS = 2, 16
NW = NC * NS
RPW = N // NW
WPB = S // RPW
G = 32
NCHUNK = RPW // G
LANES = 16
KSCALE = math.sqrt(D)
NOSEP = 2 * S


def _prep_body(pos_ref, seg_ref, out_ref):
    g = pl.program_id(0)
    seg_row = jnp.where(g >= 8, seg_ref[1], seg_ref[0])
    out_ref[...] = (pos_ref[...] + seg_row[None, :]) * KSCALE


def _make_poskc(pos_table, seg_table):
    return pl.pallas_call(
        _prep_body,
        grid=(16,),
        in_specs=[
            pl.BlockSpec((S // 8, D), lambda g: (g % 8, 0)),
            pl.BlockSpec((2, D), lambda g: (0, 0)),
        ],
        out_specs=pl.BlockSpec((S // 8, D), lambda g: (g, 0)),
        out_shape=jax.ShapeDtypeStruct((2 * S, D), jnp.float32),
    )(pos_table, seg_table)


_SC_CP = pltpu.CompilerParams()
if "needs_layout_passes" in pltpu.CompilerParams.__dataclass_fields__:
    _SC_CP = dataclasses.replace(_SC_CP, needs_layout_passes=False)


@functools.partial(
    pl.kernel,
    out_type=jax.ShapeDtypeStruct((N, D), jnp.float32),
    compiler_params=_SC_CP,
    mesh=plsc.VectorSubcoreMesh(core_axis_name="c", subcore_axis_name="s"),
    scratch_types=[
        pltpu.VMEM((LANES,), jnp.int32),
        pltpu.VMEM((S,), jnp.int32),
        pltpu.VMEM((RPW,), jnp.int32),
        pltpu.VMEM((LANES,), jnp.int32),
        pltpu.VMEM((G, D), jnp.float32),
        pltpu.VMEM((G, D), jnp.float32),
        pltpu.VMEM((G, D), jnp.float32),
        pltpu.VMEM((G, D), jnp.float32),
        pltpu.SemaphoreType.DMA,
        pltpu.SemaphoreType.DMA,
        pltpu.SemaphoreType.DMA,
        pltpu.SemaphoreType.DMA,
    ],
)
def _sc_lookup(sep_hbm, xflat_hbm, token_hbm, poskc_hbm, out_hbm,
               sep_v, xrow_v, pidx_v, minv_v, t0, p0, t1, p1,
               st0, sp0, st1, sp1):
    cid = lax.axis_index("c")
    sid = lax.axis_index("s")
    wid = sid * NC + cid
    base = wid * RPW
    bid = wid // WPB
    s0 = (wid % WPB) * RPW

    pltpu.sync_copy(sep_hbm, sep_v)
    pltpu.sync_copy(xflat_hbm.at[pl.ds(bid * S, S)], xrow_v)

    lanes = lax.iota(jnp.int32, LANES)
    minv_v[...] = jnp.full((LANES,), NOSEP, jnp.int32)

    @pl.loop(0, S // LANES)
    def _(i):
        vals = xrow_v[pl.ds(i * LANES, LANES)]
        posv = lanes + i * LANES
        cand = jnp.where(vals == sep_v[...], posv, NOSEP)
        minv_v[...] = jnp.minimum(minv_v[...], cand)

    p_first = jnp.min(minv_v[...])

    @pl.loop(0, RPW // LANES)
    def _(i):
        posv = lanes + (s0 + i * LANES)
        pidx_v[pl.ds(i * LANES, LANES)] = jnp.where(
            posv >= p_first, posv + S, posv)

    slots = ((t0, p0, st0, sp0), (t1, p1, st1, sp1))

    def issue(c, slot):
        tb, pb, st, sp = slot
        cpy_t = pltpu.async_copy(
            token_hbm.at[xrow_v.at[pl.ds(s0 + c * G, G)]], tb, st)
        cpy_p = pltpu.async_copy(
            poskc_hbm.at[pidx_v.at[pl.ds(c * G, G)]], pb, sp)
        return cpy_t, cpy_p

    pend = issue(0, slots[0])
    for c in range(NCHUNK):
        nxt = issue(c + 1, slots[(c + 1) % 2]) if c + 1 < NCHUNK else None
        cpy_t, cpy_p = pend
        cpy_t.wait()
        cpy_p.wait()
        tb, pb = slots[c % 2][0], slots[c % 2][1]

        @pl.loop(0, G)
        def _(j, tb=tb, pb=pb):
            for c2 in range(D // LANES):
                sl = pl.ds(c2 * LANES, LANES)
                tb[j, sl] = tb[j, sl] * KSCALE + pb[j, sl]

        pltpu.sync_copy(tb, out_hbm.at[pl.ds(base + c * G, G)])
        pend = nxt


def kernel(x, sep_token, token_table, pos_table, seg_table):
    poskc = _make_poskc(pos_table, seg_table)
    xflat = x.reshape(N)
    sep_vec = jnp.full((LANES,), sep_token, jnp.int32)
    out = _sc_lookup(sep_vec, xflat, token_table, poskc)
    return out.reshape(B, S, D)
